# baseline (speedup 1.0000x reference)
import functools

import jax
import jax.numpy as jnp
from jax import lax
from jax.experimental import pallas as pl
from jax.experimental.pallas import tpu as pltpu
from jax.experimental.pallas import tpu_sc as plsc

_TVS = 8192
_TVW = 4096
_LOG2E = 1.4426950408889634
_LN2 = 0.6931471805599453

_CONTRACT0 = (((0,), (0,)), ((), ()))


@functools.lru_cache(maxsize=None)
def _make_sc_gather(V, D, B):
    info = plsc.get_sparse_core_info()
    NC, NS, L = info.num_cores, info.num_subcores, info.num_lanes
    NW = NC * NS
    assert B % (L * NW) == 0 and D == L and 128 % D == 0
    b_per_w = B // NW
    rows_per_line = 128 // D
    mesh = plsc.VectorSubcoreMesh(core_axis_name="c", subcore_axis_name="s")

    @functools.partial(
        pl.kernel,
        mesh=mesh,
        out_type=jax.ShapeDtypeStruct((B, D), jnp.float32),
        scratch_types=[
            pltpu.VMEM((b_per_w,), jnp.int32),
            pltpu.VMEM((b_per_w,), jnp.int32),
            pltpu.VMEM((b_per_w, 128), jnp.float32),
            pltpu.VMEM((b_per_w, D), jnp.float32),
            pltpu.SemaphoreType.DMA,
        ],
        compiler_params=pltpu.CompilerParams(needs_layout_passes=False),
    )
    def gather_kernel(idx_hbm, table_hbm, out_hbm, idx_v, line_v, rows_v,
                      x_v, sem):
        wid = lax.axis_index("s") * NC + lax.axis_index("c")
        base = wid * b_per_w
        pltpu.sync_copy(idx_hbm.at[pl.ds(base, b_per_w)], idx_v)
        for g in range(b_per_w // L):
            iv = idx_v[pl.ds(g * L, L)]
            line_v[pl.ds(g * L, L)] = lax.div(iv, rows_per_line)
        pltpu.async_copy(table_hbm.at[line_v], rows_v, sem).wait()
        for g in range(b_per_w // L):
            iv = idx_v[pl.ds(g * L, L)]
            off = lax.rem(iv, rows_per_line) * D
            row = lax.iota(jnp.int32, L) + g * L
            for k in range(D):
                vals = plsc.load_gather(rows_v, [row, off + k])
                plsc.store_scatter(
                    x_v, [row, jnp.full((L,), k, jnp.int32)], vals
                )
        pltpu.sync_copy(x_v, out_hbm.at[pl.ds(base, b_per_w)])

    return gather_kernel


def _haug(xT_ref, W1aug_ref, B):
    ones = jnp.full((1, B), 1.0, dtype=jnp.float32)
    xaug = jnp.concatenate([xT_ref[...], ones], axis=0)
    hT = jnp.maximum(
        lax.dot_general(W1aug_ref[...], xaug, _CONTRACT0,
                        preferred_element_type=jnp.float32),
        0.0,
    )
    return jnp.concatenate([hT, ones], axis=0)


def _stats_body(xT_ref, W1aug_ref, W2_ref, b2_ref, lse_ref, m_scr, s_scr,
                *, V, nj, B):
    j = pl.program_id(0)

    @pl.when(j == 0)
    def _init():
        m_scr[...] = jnp.full_like(m_scr, -jnp.inf)
        s_scr[...] = jnp.zeros_like(s_scr)

    haug_s = _haug(xT_ref, W1aug_ref, B) * _LOG2E
    mask = j * _TVS + lax.broadcasted_iota(jnp.int32, (1, _TVS), 1) < V
    w2m = jnp.where(mask, W2_ref[...], 0.0)
    b2m = jnp.where(mask, b2_ref[...], -jnp.inf)
    w2aug = jnp.concatenate([w2m, b2m], axis=0)
    logits2 = lax.dot_general(w2aug, haug_s, _CONTRACT0,
                              preferred_element_type=jnp.float32)

    m_old = m_scr[...]
    tmax = jnp.max(logits2, axis=0, keepdims=True)
    m_new = jnp.maximum(m_old, tmax)
    p = jnp.exp2(logits2 - m_new[0:1, :])
    s_new = s_scr[...] * jnp.exp2(m_old - m_new) + jnp.sum(
        p, axis=0, keepdims=True
    )
    m_scr[...] = m_new
    s_scr[...] = s_new

    @pl.when(j == nj - 1)
    def _fin():
        lse_ref[...] = m_scr[...] * _LN2 + jnp.log(s_scr[...])


def _write_body(xT_ref, W1aug_ref, W2_ref, b2_ref, lse_ref, out_ref, *, B):
    haug = _haug(xT_ref, W1aug_ref, B)
    w2aug = jnp.concatenate([W2_ref[...], b2_ref[...]], axis=0)
    logits_t = lax.dot_general(w2aug, haug, _CONTRACT0,
                               preferred_element_type=jnp.float32)
    out_ref[...] = logits_t - lse_ref[0:1, :]


def _mlp_logsoftmax(x, W1, b1, W2, b2):
    B, E = x.shape
    H, V = W2.shape
    njs = pl.cdiv(V, _TVS)
    njw = pl.cdiv(V, _TVW)
    xT = x.T
    W1aug = jnp.concatenate([W1, b1.reshape(1, H)], axis=0)
    b2r = b2.reshape(1, V)

    full = lambda shape: pl.BlockSpec(shape, lambda j: (0, 0))
    common = lambda tv: [
        full((E, B)),
        full((E + 1, H)),
        pl.BlockSpec((H, tv), lambda j: (0, j)),
        pl.BlockSpec((1, tv), lambda j: (0, j)),
    ]
    seq = pltpu.CompilerParams(dimension_semantics=("arbitrary",))

    lse = pl.pallas_call(
        functools.partial(_stats_body, V=V, nj=njs, B=B),
        grid=(njs,),
        in_specs=common(_TVS),
        out_specs=full((8, B)),
        out_shape=jax.ShapeDtypeStruct((8, B), jnp.float32),
        scratch_shapes=[
            pltpu.VMEM((8, B), jnp.float32),
            pltpu.VMEM((8, B), jnp.float32),
        ],
        compiler_params=seq,
    )(xT, W1aug, W2, b2r)

    out_t = pl.pallas_call(
        functools.partial(_write_body, B=B),
        grid=(njw,),
        in_specs=common(_TVW) + [full((8, B))],
        out_specs=pl.BlockSpec((_TVW, B), lambda j: (j, 0)),
        out_shape=jax.ShapeDtypeStruct((V, B), jnp.float32),
        compiler_params=seq,
    )(xT, W1aug, W2, b2r, lse)
    return out_t.T


def kernel(inputs, emb, W1, b1, W2, b2):
    V, E = emb.shape
    (B,) = inputs.shape
    table = emb.reshape(V * E // 128, 128)
    x = _make_sc_gather(V, E, B)(inputs.astype(jnp.int32), table)
    return _mlp_logsoftmax(x, W1, b1, W2, b2)


# speedup vs baseline: 1.0164x; 1.0164x over previous
import functools

import jax
import jax.numpy as jnp
from jax import lax
from jax.experimental import pallas as pl
from jax.experimental.pallas import tpu as pltpu
from jax.experimental.pallas import tpu_sc as plsc

_TVS = 8192
_TVW = 4096
_LOG2E = 1.4426950408889634
_LN2 = 0.6931471805599453

_CONTRACT0 = (((0,), (0,)), ((), ()))


@functools.lru_cache(maxsize=None)
def _make_sc_gather(V, D, B):
    info = plsc.get_sparse_core_info()
    NC, NS, L = info.num_cores, info.num_subcores, info.num_lanes
    NW = NC * NS
    assert B % (L * NW) == 0 and D == L and 128 % D == 0
    b_per_w = B // NW
    rows_per_line = 128 // D
    mesh = plsc.VectorSubcoreMesh(core_axis_name="c", subcore_axis_name="s")

    @functools.partial(
        pl.kernel,
        mesh=mesh,
        out_type=jax.ShapeDtypeStruct((B, D), jnp.float32),
        scratch_types=[
            pltpu.VMEM((b_per_w,), jnp.int32),
            pltpu.VMEM((b_per_w,), jnp.int32),
            pltpu.VMEM((b_per_w, 128), jnp.float32),
            pltpu.VMEM((b_per_w, D), jnp.float32),
            pltpu.SemaphoreType.DMA,
        ],
        compiler_params=pltpu.CompilerParams(needs_layout_passes=False),
    )
    def gather_kernel(idx_hbm, table_hbm, out_hbm, idx_v, line_v, rows_v,
                      x_v, sem):
        wid = lax.axis_index("s") * NC + lax.axis_index("c")
        base = wid * b_per_w
        pltpu.sync_copy(idx_hbm.at[pl.ds(base, b_per_w)], idx_v)
        for g in range(b_per_w // L):
            iv = idx_v[pl.ds(g * L, L)]
            line_v[pl.ds(g * L, L)] = lax.div(iv, rows_per_line)
        pltpu.async_copy(table_hbm.at[line_v], rows_v, sem).wait()
        for g in range(b_per_w // L):
            iv = idx_v[pl.ds(g * L, L)]
            off = lax.rem(iv, rows_per_line) * D
            row = lax.iota(jnp.int32, L) + g * L
            for k in range(D):
                vals = plsc.load_gather(rows_v, [row, off + k])
                plsc.store_scatter(
                    x_v, [row, jnp.full((L,), k, jnp.int32)], vals
                )
        pltpu.sync_copy(x_v, out_hbm.at[pl.ds(base, b_per_w)])

    return gather_kernel


def _haug(xT_ref, W1aug_ref, B):
    ones = jnp.full((1, B), 1.0, dtype=jnp.float32)
    xaug = jnp.concatenate([xT_ref[...], ones], axis=0)
    hT = jnp.maximum(
        lax.dot_general(W1aug_ref[...], xaug, _CONTRACT0,
                        preferred_element_type=jnp.float32),
        0.0,
    )
    return jnp.concatenate([hT, ones], axis=0)


def _stats_body(xT_ref, W1aug_ref, W2_ref, b2_ref, lse_ref, m_scr, s_scr,
                *, V, nj, B):
    j = pl.program_id(0)

    @pl.when(j == 0)
    def _init():
        m_scr[...] = jnp.zeros_like(m_scr)
        s_scr[...] = jnp.zeros_like(s_scr)

    haug_s = jnp.concatenate(
        [_haug(xT_ref, W1aug_ref, B) * _LOG2E, -m_scr[0:1, :]], axis=0
    )
    mask = j * _TVS + lax.broadcasted_iota(jnp.int32, (1, _TVS), 1) < V
    w2m = jnp.where(mask, W2_ref[...], 0.0)
    b2m = jnp.where(mask, b2_ref[...], -jnp.inf)
    ones = jnp.full((1, _TVS), 1.0, dtype=jnp.float32)
    w2aug = jnp.concatenate([w2m, b2m, ones], axis=0)
    t = lax.dot_general(w2aug, haug_s, _CONTRACT0,
                        preferred_element_type=jnp.float32)

    tmax = jnp.max(t, axis=0, keepdims=True)
    rescale = jnp.logical_or(j == 0, jnp.max(tmax) > 60.0)

    @pl.when(jnp.logical_not(rescale))
    def _fast():
        s_scr[0:1, :] = s_scr[0:1, :] + jnp.sum(jnp.exp2(t), axis=0,
                                                keepdims=True)

    @pl.when(rescale)
    def _slow():
        shift = jnp.where(j == 0, tmax, jnp.maximum(tmax, 0.0))
        prev = jnp.where(j == 0, 0.0, s_scr[0:1, :] * jnp.exp2(-shift))
        s_scr[0:1, :] = prev + jnp.sum(
            jnp.exp2(t - shift), axis=0, keepdims=True
        )
        m_scr[...] = m_scr[...] + shift

    @pl.when(j == nj - 1)
    def _fin():
        lse_ref[...] = (m_scr[...] + jnp.log2(s_scr[...])) * _LN2


def _write_body(xT_ref, W1aug_ref, W2_ref, b2_ref, lse_ref, out_ref, *, B):
    haug = jnp.concatenate(
        [_haug(xT_ref, W1aug_ref, B), -lse_ref[0:1, :]], axis=0
    )
    ones = jnp.full((1, out_ref.shape[0]), 1.0, dtype=jnp.float32)
    w2aug = jnp.concatenate([W2_ref[...], b2_ref[...], ones], axis=0)
    out_ref[...] = lax.dot_general(w2aug, haug, _CONTRACT0,
                                   preferred_element_type=jnp.float32)


def _mlp_logsoftmax(xT, W1, b1, W2, b2):
    E, B = xT.shape
    H, V = W2.shape
    njs = pl.cdiv(V, _TVS)
    njw = pl.cdiv(V, _TVW)
    W1aug = jnp.concatenate([W1, b1.reshape(1, H)], axis=0)
    b2r = b2.reshape(1, V)

    full = lambda shape: pl.BlockSpec(shape, lambda j: (0, 0))
    common = lambda tv: [
        full((E, B)),
        full((E + 1, H)),
        pl.BlockSpec((H, tv), lambda j: (0, j)),
        pl.BlockSpec((1, tv), lambda j: (0, j)),
    ]
    seq = pltpu.CompilerParams(dimension_semantics=("arbitrary",))

    lse = pl.pallas_call(
        functools.partial(_stats_body, V=V, nj=njs, B=B),
        grid=(njs,),
        in_specs=common(_TVS),
        out_specs=full((8, B)),
        out_shape=jax.ShapeDtypeStruct((8, B), jnp.float32),
        scratch_shapes=[
            pltpu.VMEM((8, B), jnp.float32),
            pltpu.VMEM((8, B), jnp.float32),
        ],
        compiler_params=seq,
    )(xT, W1aug, W2, b2r)

    out_t = pl.pallas_call(
        functools.partial(_write_body, B=B),
        grid=(njw,),
        in_specs=common(_TVW) + [full((8, B))],
        out_specs=pl.BlockSpec((_TVW, B), lambda j: (j, 0)),
        out_shape=jax.ShapeDtypeStruct((V, B), jnp.float32),
        compiler_params=seq,
    )(xT, W1aug, W2, b2r, lse)
    return out_t.T


def kernel(inputs, emb, W1, b1, W2, b2):
    V, E = emb.shape
    (B,) = inputs.shape
    table = emb.reshape(V * E // 128, 128)
    x = _make_sc_gather(V, E, B)(inputs.astype(jnp.int32), table)
    return _mlp_logsoftmax(x.T, W1, b1, W2, b2)


# speedup vs baseline: 1.0172x; 1.0008x over previous
import functools

import jax
import jax.numpy as jnp
from jax import lax
from jax.experimental import pallas as pl
from jax.experimental.pallas import tpu as pltpu
from jax.experimental.pallas import tpu_sc as plsc

_TVS = 8192
_TVW = 4096
_LOG2E = 1.4426950408889634
_LN2 = 0.6931471805599453

_CONTRACT0 = (((0,), (0,)), ((), ()))


@functools.lru_cache(maxsize=None)
def _make_sc_gather(V, D, B):
    info = plsc.get_sparse_core_info()
    NC, NS, L = info.num_cores, info.num_subcores, info.num_lanes
    NW = NC * NS
    assert B % (L * NW) == 0 and D == L and 128 % D == 0
    b_per_w = B // NW
    rows_per_line = 128 // D
    mesh = plsc.VectorSubcoreMesh(core_axis_name="c", subcore_axis_name="s")

    @functools.partial(
        pl.kernel,
        mesh=mesh,
        out_type=jax.ShapeDtypeStruct((B, D), jnp.float32),
        scratch_types=[
            pltpu.VMEM((b_per_w,), jnp.int32),
            pltpu.VMEM((b_per_w,), jnp.int32),
            pltpu.VMEM((b_per_w, 128), jnp.float32),
            pltpu.VMEM((b_per_w, D), jnp.float32),
            pltpu.SemaphoreType.DMA,
        ],
        compiler_params=pltpu.CompilerParams(needs_layout_passes=False),
    )
    def gather_kernel(idx_hbm, table_hbm, out_hbm, idx_v, line_v, rows_v,
                      x_v, sem):
        wid = lax.axis_index("s") * NC + lax.axis_index("c")
        base = wid * b_per_w
        pltpu.sync_copy(idx_hbm.at[pl.ds(base, b_per_w)], idx_v)
        for g in range(b_per_w // L):
            iv = idx_v[pl.ds(g * L, L)]
            line_v[pl.ds(g * L, L)] = lax.div(iv, rows_per_line)
        pltpu.async_copy(table_hbm.at[line_v], rows_v, sem).wait()
        for g in range(b_per_w // L):
            iv = idx_v[pl.ds(g * L, L)]
            off = lax.rem(iv, rows_per_line) * D
            row = lax.iota(jnp.int32, L) + g * L
            for k in range(D):
                vals = plsc.load_gather(rows_v, [row, off + k])
                plsc.store_scatter(
                    x_v, [row, jnp.full((L,), k, jnp.int32)], vals
                )
        pltpu.sync_copy(x_v, out_hbm.at[pl.ds(base, b_per_w)])

    return gather_kernel


def _haug(xT_ref, W1aug_ref, B):
    ones = jnp.full((1, B), 1.0, dtype=jnp.float32)
    xaug = jnp.concatenate([xT_ref[...], ones], axis=0)
    hT = jnp.maximum(
        lax.dot_general(W1aug_ref[...], xaug, _CONTRACT0,
                        preferred_element_type=jnp.float32),
        0.0,
    )
    return jnp.concatenate([hT, ones], axis=0)


def _stats_body(xT_ref, W1aug_ref, W2_ref, b2_ref, lse_ref, m_scr, s_scr,
                *, V, nj, B):
    j = pl.program_id(0)

    @pl.when(j == 0)
    def _init():
        m_scr[...] = jnp.zeros_like(m_scr)
        s_scr[...] = jnp.zeros_like(s_scr)

    haug_s = jnp.concatenate(
        [_haug(xT_ref, W1aug_ref, B) * _LOG2E, -m_scr[0:1, :]], axis=0
    )
    mask = j * _TVS + lax.broadcasted_iota(jnp.int32, (1, _TVS), 1) < V
    w2m = jnp.where(mask, W2_ref[...], 0.0)
    b2m = jnp.where(mask, b2_ref[...], -jnp.inf)
    ones = jnp.full((1, _TVS), 1.0, dtype=jnp.float32)
    w2aug = jnp.concatenate([w2m, b2m, ones], axis=0)
    t = lax.dot_general(w2aug, haug_s, _CONTRACT0,
                        preferred_element_type=jnp.float32)

    tmax = jnp.max(t, axis=0, keepdims=True)
    rescale = jnp.logical_or(j == 0, jnp.max(tmax) > 60.0)

    @pl.when(jnp.logical_not(rescale))
    def _fast():
        s_scr[0:1, :] = s_scr[0:1, :] + jnp.sum(jnp.exp2(t), axis=0,
                                                keepdims=True)

    @pl.when(rescale)
    def _slow():
        shift = jnp.where(j == 0, tmax, jnp.maximum(tmax, 0.0))
        prev = jnp.where(j == 0, 0.0, s_scr[0:1, :] * jnp.exp2(-shift))
        s_scr[0:1, :] = prev + jnp.sum(
            jnp.exp2(t - shift), axis=0, keepdims=True
        )
        m_scr[...] = m_scr[...] + shift

    @pl.when(j == nj - 1)
    def _fin():
        lse_ref[...] = (m_scr[...] + jnp.log2(s_scr[...])) * _LN2


def _write_body(xT_ref, W1aug_ref, W2_ref, b2_ref, lse_ref, out_ref, *, B):
    haug = jnp.concatenate(
        [_haug(xT_ref, W1aug_ref, B), -lse_ref[0:1, :]], axis=0
    )
    ones = jnp.full((1, out_ref.shape[0]), 1.0, dtype=jnp.float32)
    w2aug = jnp.concatenate([W2_ref[...], b2_ref[...], ones], axis=0)
    out_ref[...] = lax.dot_general(w2aug, haug, _CONTRACT0,
                                   preferred_element_type=jnp.float32)


def _mlp_logsoftmax(xT, W1, b1, W2, b2):
    E, B = xT.shape
    H, V = W2.shape
    njs = pl.cdiv(V, _TVS)
    njw = pl.cdiv(V, _TVW)
    W1aug = jnp.concatenate([W1, b1.reshape(1, H)], axis=0)
    b2r = b2.reshape(1, V)

    full = lambda shape: pl.BlockSpec(shape, lambda j: (0, 0))
    common = lambda tv: [
        full((E, B)),
        full((E + 1, H)),
        pl.BlockSpec((H, tv), lambda j: (0, j)),
        pl.BlockSpec((1, tv), lambda j: (0, j)),
    ]
    seq = pltpu.CompilerParams(dimension_semantics=("arbitrary",))
    par = pltpu.CompilerParams(dimension_semantics=("parallel",))

    lse = pl.pallas_call(
        functools.partial(_stats_body, V=V, nj=njs, B=B),
        grid=(njs,),
        in_specs=common(_TVS),
        out_specs=full((8, B)),
        out_shape=jax.ShapeDtypeStruct((8, B), jnp.float32),
        scratch_shapes=[
            pltpu.VMEM((8, B), jnp.float32),
            pltpu.VMEM((8, B), jnp.float32),
        ],
        compiler_params=seq,
    )(xT, W1aug, W2, b2r)

    out_t = pl.pallas_call(
        functools.partial(_write_body, B=B),
        grid=(njw,),
        in_specs=common(_TVW) + [full((8, B))],
        out_specs=pl.BlockSpec((_TVW, B), lambda j: (j, 0)),
        out_shape=jax.ShapeDtypeStruct((V, B), jnp.float32),
        compiler_params=par,
    )(xT, W1aug, W2, b2r, lse)
    return out_t.T


def kernel(inputs, emb, W1, b1, W2, b2):
    V, E = emb.shape
    (B,) = inputs.shape
    table = emb.reshape(V * E // 128, 128)
    x = _make_sc_gather(V, E, B)(inputs.astype(jnp.int32), table)
    return _mlp_logsoftmax(x.T, W1, b1, W2, b2)
